# Initial kernel scaffold; baseline (speedup 1.0000x reference)
import functools

import jax
import jax.numpy as jnp
from jax import lax
from jax.experimental import pallas as pl
from jax.experimental.pallas import tpu as pltpu
from jax.experimental.pallas import tpu_sc as plsc

_HALF = 32
_LANES = 16
_CHUNK = 2048
_GRP = _CHUNK // 128
_NLAYERS = 3


def _sc_propagate(cat_tab, src2, dst2, vals, n_nodes, per_tile, n_chunks):
    mesh = plsc.VectorSubcoreMesh(core_axis_name="c", subcore_axis_name="s")
    n_sub = mesh.num_subcores
    rows_per_tile = n_nodes // n_sub
    zrows = 625
    out_sds = jax.ShapeDtypeStruct((2 * n_nodes, _HALF), jnp.float32)

    @functools.partial(
        pl.kernel,
        out_type=(out_sds, out_sds, out_sds),
        mesh=mesh,
        scratch_types=[
            pltpu.VMEM((_CHUNK, _HALF), jnp.float32),
            pltpu.VMEM((_GRP, 128), jnp.int32),
            pltpu.VMEM((_GRP, 128), jnp.int32),
            pltpu.VMEM((_CHUNK,), jnp.float32),
            pltpu.VMEM((625, _HALF), jnp.float32),
            pltpu.VMEM_SHARED((n_nodes, _HALF), jnp.float32),
            pltpu.SemaphoreType.DMA,
            pltpu.SemaphoreType.DMA,
        ],
    )
    def body(tab, srcr, dstr, valr, out1, out2, out3,
             rows, sidx, didx, vbuf, zbuf, acc, gsem, ssem):
        c = lax.axis_index("c")
        s = lax.axis_index("s")
        lane = lax.broadcasted_iota(jnp.int32, (_LANES,), 0)
        z16 = jnp.zeros((_LANES,), jnp.float32)

        def zfill(i, _):
            ridx = jnp.full((_LANES,), 0, jnp.int32) + i
            plsc.store_scatter(zbuf, [ridx, lane], z16)
            plsc.store_scatter(zbuf, [ridx, lane + _LANES], z16)
            return 0
        lax.fori_loop(0, zrows, zfill, 0)

        outs = (out1, out2, out3)
        for l in range(_NLAYERS):
            srctab = tab if l == 0 else outs[l - 1]
            for j in range(rows_per_tile // zrows):
                pltpu.sync_copy(zbuf, acc.at[pl.ds(s * rows_per_tile + j * zrows, zrows)])
            plsc.subcore_barrier()

            def chunk_body(ch, _, srctab=srctab):
                row0 = s * (per_tile // 128) + ch * _GRP
                pltpu.sync_copy(srcr.at[c, pl.ds(row0, _GRP)], sidx)
                pltpu.sync_copy(dstr.at[pl.ds(row0, _GRP)], didx)
                pltpu.sync_copy(valr.at[pl.ds(s * per_tile + ch * _CHUNK, _CHUNK)], vbuf)
                gds = [
                    pltpu.async_copy(srctab.at[sidx.at[g]],
                                     rows.at[pl.ds(g * 128, 128)], gsem)
                    for g in range(_GRP)
                ]
                for d_ in gds:
                    d_.wait()

                def grp(i, _):
                    v = vbuf[pl.ds(i * _LANES, _LANES)]
                    ridx = lane + i * _LANES
                    for dcol in range(_HALF):
                        cidx = jnp.full((_LANES,), dcol, jnp.int32)
                        gv = plsc.load_gather(rows, [ridx, cidx])
                        plsc.store_scatter(rows, [ridx, cidx], gv * v)
                    return 0
                lax.fori_loop(0, _CHUNK // _LANES, grp, 0)

                sds = [
                    pltpu.async_copy(rows.at[pl.ds(g * 128, 128)],
                                     acc.at[didx.at[g]], ssem, add=True)
                    for g in range(_GRP)
                ]
                for d_ in sds:
                    d_.wait()
                return 0
            lax.fori_loop(0, n_chunks, chunk_body, 0)
            plsc.subcore_barrier()

            r0 = s * rows_per_tile
            pltpu.sync_copy(acc.at[pl.ds(r0, rows_per_tile)],
                            outs[l].at[pl.ds(c * n_nodes + r0, rows_per_tile)])
            plsc.subcore_barrier()

    return body(cat_tab, src2, dst2, vals)


def _tc_finalize(cat_tab, o1, o2, o3, n_nodes):
    blk = 1000
    nblk = n_nodes // blk
    half_off = n_nodes // blk

    def fin(t0, t1, a0, a1, b0, b1, c0, c1, out):
        m0 = (t0[...] + a0[...] + b0[...] + c0[...]) * 0.25
        m1 = (t1[...] + a1[...] + b1[...] + c1[...]) * 0.25
        ns = jnp.sum(m0 * m0, axis=1, keepdims=True) + jnp.sum(m1 * m1, axis=1, keepdims=True)
        inv = 1.0 / jnp.maximum(jnp.sqrt(ns), 1e-12)
        out[:, :_HALF] = m0 * inv
        out[:, _HALF:] = m1 * inv

    spec0 = pl.BlockSpec((blk, _HALF), lambda i: (i, 0))
    spec1 = pl.BlockSpec((blk, _HALF), lambda i: (i + half_off, 0))
    return pl.pallas_call(
        fin,
        grid=(nblk,),
        in_specs=[spec0, spec1, spec0, spec1, spec0, spec1, spec0, spec1],
        out_specs=pl.BlockSpec((blk, 2 * _HALF), lambda i: (i, 0)),
        out_shape=jax.ShapeDtypeStruct((n_nodes, 2 * _HALF), jnp.float32),
    )(cat_tab, cat_tab, o1, o1, o2, o2, o3, o3)


def kernel(user_emb, item_emb, edge_vals, edge_index):
    n_users = user_emb.shape[0]
    n_nodes = n_users + item_emb.shape[0]
    n_edges = edge_vals.shape[0]
    mesh = plsc.VectorSubcoreMesh(core_axis_name="c", subcore_axis_name="s")
    n_tiles = mesh.num_subcores
    e_pad = ((n_edges + n_tiles * _CHUNK - 1) // (n_tiles * _CHUNK)) * (n_tiles * _CHUNK)
    per_tile = e_pad // n_tiles
    n_chunks = per_tile // _CHUNK

    all_emb = jnp.concatenate([user_emb, item_emb], axis=0).astype(jnp.float32)
    cat_tab = jnp.concatenate([all_emb[:, :_HALF], all_emb[:, _HALF:]], axis=0)

    src = edge_index[0].astype(jnp.int32)
    dst = edge_index[1].astype(jnp.int32)
    vals = edge_vals.astype(jnp.float32)
    pad = e_pad - n_edges
    src = jnp.concatenate([src, jnp.zeros((pad,), jnp.int32)])
    dst = jnp.concatenate([dst, jnp.zeros((pad,), jnp.int32)])
    vals = jnp.concatenate([vals, jnp.zeros((pad,), jnp.float32)])
    src2 = jnp.stack([src, src + n_nodes]).reshape(2, e_pad // 128, 128)
    dst2 = dst.reshape(e_pad // 128, 128)

    o1, o2, o3 = _sc_propagate(cat_tab, src2, dst2, vals, n_nodes, per_tile, n_chunks)
    res = _tc_finalize(cat_tab, o1, o2, o3, n_nodes)
    return res[:n_users], res[n_users:]


# speedup vs baseline: 5.1589x; 5.1589x over previous
import functools

import jax
import jax.numpy as jnp
from jax import lax
from jax.experimental import pallas as pl
from jax.experimental.pallas import tpu as pltpu
from jax.experimental.pallas import tpu_sc as plsc

_HALF = 32
_LANES = 16
_CHUNK = 2048
_SUB = 512
_GRP = _CHUNK // 128
_SGRP = _SUB // 128
_NLAYERS = 3
_ZROWS = 184


def _sc_propagate(cat_tab, src2, dst2, vals, n_pad, per_tile, n_chunks):
    mesh = plsc.VectorSubcoreMesh(core_axis_name="c", subcore_axis_name="s")
    n_sub = mesh.num_subcores
    rows_per_tile = n_pad // n_sub
    out_sds = jax.ShapeDtypeStruct((2 * n_pad, _HALF), jnp.float32)

    @functools.partial(
        pl.kernel,
        out_type=(out_sds, out_sds, out_sds),
        mesh=mesh,
        compiler_params=pltpu.CompilerParams(use_tc_tiling_on_sc=False),
        scratch_types=[
            pltpu.VMEM((_SUB, _HALF), jnp.float32),
            pltpu.VMEM((_GRP, 128), jnp.int32),
            pltpu.VMEM((_GRP, 128), jnp.int32),
            pltpu.VMEM((_CHUNK,), jnp.float32),
            pltpu.VMEM_SHARED((n_pad, _HALF), jnp.float32),
            pltpu.SemaphoreType.DMA,
            pltpu.SemaphoreType.DMA,
        ],
    )
    def body(tab, srcr, dstr, valr, out1, out2, out3,
             rows, sidx, didx, vbuf, acc, gsem, ssem):
        c = lax.axis_index("c")
        s = lax.axis_index("s")
        z16 = jnp.zeros((_LANES,), jnp.float32)

        outs = (out1, out2, out3)
        for l in range(_NLAYERS):
            srctab = tab if l == 0 else outs[l - 1]
            def zfill(i, _):
                rows[i, pl.ds(0, _LANES)] = z16
                rows[i, pl.ds(_LANES, _LANES)] = z16
                return 0
            lax.fori_loop(0, _ZROWS, zfill, 0)
            for j in range(rows_per_tile // _ZROWS):
                z0 = pl.multiple_of(s * rows_per_tile + j * _ZROWS, 8)
                pltpu.sync_copy(rows.at[pl.ds(0, _ZROWS)], acc.at[pl.ds(z0, _ZROWS)])
            plsc.subcore_barrier()

            def chunk_body(ch, _, srctab=srctab):
                row0 = pl.multiple_of(s * (per_tile // 128) + ch * _GRP, 8)
                v0 = pl.multiple_of(s * per_tile + ch * _CHUNK, 8)
                pltpu.sync_copy(srcr.at[c, pl.ds(row0, _GRP)], sidx)
                pltpu.sync_copy(dstr.at[pl.ds(row0, _GRP)], didx)
                pltpu.sync_copy(valr.at[pl.ds(v0, _CHUNK)], vbuf)
                for sb in range(_CHUNK // _SUB):
                    gds = [
                        pltpu.async_copy(srctab.at[sidx.at[sb * _SGRP + g]],
                                         rows.at[pl.ds(g * 128, 128)], gsem)
                        for g in range(_SGRP)
                    ]
                    for d_ in gds:
                        d_.wait()

                    def grp(i, _, sb=sb):
                        v = vbuf[pl.ds(sb * _SUB + i * _LANES, _LANES)]
                        for j in range(_LANES):
                            e = i * _LANES + j
                            bc = jnp.take_along_axis(
                                v, jnp.full((_LANES,), j, jnp.int32), axis=0)
                            rows[e, pl.ds(0, _LANES)] = (
                                rows[e, pl.ds(0, _LANES)] * bc)
                            rows[e, pl.ds(_LANES, _LANES)] = (
                                rows[e, pl.ds(_LANES, _LANES)] * bc)
                        return 0
                    lax.fori_loop(0, _SUB // _LANES, grp, 0)

                    sds = [
                        pltpu.async_copy(rows.at[pl.ds(g * 128, 128)],
                                         acc.at[didx.at[sb * _SGRP + g]],
                                         ssem, add=True)
                        for g in range(_SGRP)
                    ]
                    for d_ in sds:
                        d_.wait()
                return 0
            lax.fori_loop(0, n_chunks, chunk_body, 0)
            plsc.subcore_barrier()

            r0 = pl.multiple_of(s * rows_per_tile, 8)
            w0 = pl.multiple_of(c * n_pad + s * rows_per_tile, 8)
            pltpu.sync_copy(acc.at[pl.ds(r0, rows_per_tile)],
                            outs[l].at[pl.ds(w0, rows_per_tile)])
            plsc.subcore_barrier()

    return body(cat_tab, src2, dst2, vals)


def _tc_finalize(cat_tab, o1, o2, o3, n_pad):
    blk = 544
    nblk = n_pad // blk
    half_off = n_pad // blk

    def fin(t0, t1, a0, a1, b0, b1, c0, c1, out):
        m0 = (t0[...] + a0[...] + b0[...] + c0[...]) * 0.25
        m1 = (t1[...] + a1[...] + b1[...] + c1[...]) * 0.25
        ns = jnp.sum(m0 * m0, axis=1, keepdims=True) + jnp.sum(m1 * m1, axis=1, keepdims=True)
        inv = 1.0 / jnp.maximum(jnp.sqrt(ns), 1e-12)
        out[:, :_HALF] = m0 * inv
        out[:, _HALF:] = m1 * inv

    spec0 = pl.BlockSpec((blk, _HALF), lambda i: (i, 0))
    spec1 = pl.BlockSpec((blk, _HALF), lambda i: (i + half_off, 0))
    return pl.pallas_call(
        fin,
        grid=(nblk,),
        in_specs=[spec0, spec1, spec0, spec1, spec0, spec1, spec0, spec1],
        out_specs=pl.BlockSpec((blk, 2 * _HALF), lambda i: (i, 0)),
        out_shape=jax.ShapeDtypeStruct((n_pad, 2 * _HALF), jnp.float32),
    )(cat_tab, cat_tab, o1, o1, o2, o2, o3, o3)


def kernel(user_emb, item_emb, edge_vals, edge_index):
    n_users = user_emb.shape[0]
    n_nodes = n_users + item_emb.shape[0]
    n_edges = edge_vals.shape[0]
    mesh = plsc.VectorSubcoreMesh(core_axis_name="c", subcore_axis_name="s")
    n_tiles = mesh.num_subcores
    n_pad = ((n_nodes + 8 * n_tiles - 1) // (8 * n_tiles)) * (8 * n_tiles)
    e_pad = ((n_edges + n_tiles * _CHUNK - 1) // (n_tiles * _CHUNK)) * (n_tiles * _CHUNK)
    per_tile = e_pad // n_tiles
    n_chunks = per_tile // _CHUNK

    all_emb = jnp.concatenate([user_emb, item_emb], axis=0).astype(jnp.float32)
    all_emb = jnp.pad(all_emb, ((0, n_pad - n_nodes), (0, 0)))
    cat_tab = jnp.concatenate([all_emb[:, :_HALF], all_emb[:, _HALF:]], axis=0)

    src = edge_index[0].astype(jnp.int32)
    dst = edge_index[1].astype(jnp.int32)
    vals = edge_vals.astype(jnp.float32)
    pad = e_pad - n_edges
    src = jnp.concatenate([src, jnp.zeros((pad,), jnp.int32)])
    dst = jnp.concatenate([dst, jnp.zeros((pad,), jnp.int32)])
    vals = jnp.concatenate([vals, jnp.zeros((pad,), jnp.float32)])
    src2 = jnp.stack([src, src + n_pad]).reshape(2, e_pad // 128, 128)
    dst2 = dst.reshape(e_pad // 128, 128)

    o1, o2, o3 = _sc_propagate(cat_tab, src2, dst2, vals, n_pad, per_tile, n_chunks)
    res = _tc_finalize(cat_tab, o1, o2, o3, n_pad)
    return res[:n_users], res[n_users:n_nodes]


# speedup vs baseline: 9.9466x; 1.9281x over previous
import functools

import jax
import jax.numpy as jnp
from jax import lax
from jax.experimental import pallas as pl
from jax.experimental.pallas import tpu as pltpu
from jax.experimental.pallas import tpu_sc as plsc

_HALF = 32
_LANES = 16
_CHUNK = 1024
_SLOTS = _CHUNK // 128
_RING = 4
_NLAYERS = 3


def _sc_propagate(cat_tab, src2, dst2, vals, n_pad, per_tile):
    mesh = plsc.VectorSubcoreMesh(core_axis_name="c", subcore_axis_name="s")
    n_sub = mesh.num_subcores
    rows_per_tile = n_pad // n_sub
    n_sup = per_tile // _CHUNK
    out_sds = jax.ShapeDtypeStruct((2 * n_pad, _HALF), jnp.float32)

    @functools.partial(
        pl.kernel,
        out_type=(out_sds, out_sds, out_sds),
        mesh=mesh,
        compiler_params=pltpu.CompilerParams(use_tc_tiling_on_sc=False),
        scratch_types=[
            pltpu.VMEM((128, _HALF), jnp.float32),
            pltpu.VMEM((128, _HALF), jnp.float32),
            pltpu.VMEM((128, _HALF), jnp.float32),
            pltpu.VMEM((128, _HALF), jnp.float32),
            pltpu.VMEM((3, _SLOTS, 128), jnp.int32),
            pltpu.VMEM((3, _SLOTS, 128), jnp.int32),
            pltpu.VMEM((3, _CHUNK), jnp.float32),
            pltpu.VMEM_SHARED((n_pad, _HALF), jnp.float32),
            [pltpu.SemaphoreType.DMA] * _RING,
            [pltpu.SemaphoreType.DMA] * _RING,
            pltpu.SemaphoreType.DMA,
        ],
    )
    def body(tab, srcr, dstr, valr, out1, out2, out3,
             r0, r1, r2, r3, sidx3, didx3, vbuf3, acc, gsems, ssems, psem):
        c = lax.axis_index("c")
        s = lax.axis_index("s")
        ring = (r0, r1, r2, r3)
        z16 = jnp.zeros((_LANES,), jnp.float32)

        def stage_descs(sup):
            b = lax.rem(sup, 3)
            row0 = pl.multiple_of(s * (per_tile // 128) + sup * _SLOTS, 8)
            v0 = pl.multiple_of(s * per_tile + sup * _CHUNK, 8)
            return (
                pltpu.make_async_copy(srcr.at[c, pl.ds(row0, _SLOTS)], sidx3.at[b], psem),
                pltpu.make_async_copy(dstr.at[pl.ds(row0, _SLOTS)], didx3.at[b], psem),
                pltpu.make_async_copy(valr.at[pl.ds(v0, _CHUNK)], vbuf3.at[b], psem),
            )

        def gather_desc(b, slot, k):
            return pltpu.make_async_copy(
                tabref.at[sidx3.at[b, slot]], ring[k], gsems[k])

        def scatter_desc(b, slot, k):
            return pltpu.make_async_copy(
                ring[k], acc.at[didx3.at[b, slot]], ssems[k])

        def compute(b, slot, k):
            rbuf = ring[k]

            def grp(i, _):
                v = vbuf3[b, pl.ds(slot * 128 + i * _LANES, _LANES)]
                for j in range(_LANES):
                    e = i * _LANES + j
                    bc = jnp.take_along_axis(
                        v, jnp.full((_LANES,), j, jnp.int32), axis=0)
                    rbuf[e, pl.ds(0, _LANES)] = rbuf[e, pl.ds(0, _LANES)] * bc
                    rbuf[e, pl.ds(_LANES, _LANES)] = (
                        rbuf[e, pl.ds(_LANES, _LANES)] * bc)
                return 0
            lax.fori_loop(0, 128 // _LANES, grp, 0)

        outs = (out1, out2, out3)
        for l in range(_NLAYERS):
            tabref = tab if l == 0 else outs[l - 1]

            for d_ in stage_descs(0):
                d_.start()

            def zfill(i, _):
                r0[i, pl.ds(0, _LANES)] = z16
                r0[i, pl.ds(_LANES, _LANES)] = z16
                return 0
            lax.fori_loop(0, 128, zfill, 0)
            zbase = s * rows_per_tile
            for j in range(rows_per_tile // 128):
                z0 = pl.multiple_of(zbase + j * 128, 8)
                pltpu.sync_copy(r0, acc.at[pl.ds(z0, 128)])
            zrem = rows_per_tile % 128
            if zrem:
                z0 = pl.multiple_of(zbase + (rows_per_tile // 128) * 128, 8)
                pltpu.sync_copy(r0.at[pl.ds(0, zrem)], acc.at[pl.ds(z0, zrem)])
            plsc.subcore_barrier()

            for d_ in stage_descs(0):
                d_.wait()
            for d_ in stage_descs(1):
                d_.start()
            gather_desc(jnp.int32(0), 0, 0).start()
            gather_desc(jnp.int32(0), 1, 1).start()

            def sup_body(sup, _):
                b = lax.rem(sup, 3)
                for t in range(_SLOTS):
                    k = t % _RING
                    if t < _SLOTS - 2:
                        kk = (t + 2) % _RING
                        if t < 2:
                            @pl.when(sup > 0)
                            def _(kk=kk):
                                scatter_desc(b, 0, kk).wait()
                        else:
                            scatter_desc(b, 0, kk).wait()
                        gather_desc(b, t + 2, kk).start()
                    gather_desc(b, t, k).wait()
                    compute(b, t, k)
                    scatter_desc(b, t, k).start(add=True)

                @pl.when(sup < n_sup - 1)
                def _():
                    bn = lax.rem(sup + 1, 3)
                    for d_ in stage_descs(sup + 1):
                        d_.wait()

                    @pl.when(sup < n_sup - 2)
                    def _():
                        for d_ in stage_descs(sup + 2):
                            d_.start()
                    for t in (0, 1):
                        scatter_desc(bn, 0, t % _RING).wait()
                        gather_desc(bn, t, t % _RING).start()
                return 0
            lax.fori_loop(0, n_sup, sup_body, 0)

            for k in range(_RING):
                scatter_desc(jnp.int32(0), 0, k).wait()
            plsc.subcore_barrier()

            r0_ = pl.multiple_of(s * rows_per_tile, 8)
            w0 = pl.multiple_of(c * n_pad + s * rows_per_tile, 8)
            pltpu.sync_copy(acc.at[pl.ds(r0_, rows_per_tile)],
                            outs[l].at[pl.ds(w0, rows_per_tile)])
            plsc.subcore_barrier()

    return body(cat_tab, src2, dst2, vals)


def _tc_finalize(cat_tab, o1, o2, o3, n_pad):
    blk = 544
    nblk = n_pad // blk
    half_off = n_pad // blk

    def fin(t0, t1, a0, a1, b0, b1, c0, c1, out):
        m0 = (t0[...] + a0[...] + b0[...] + c0[...]) * 0.25
        m1 = (t1[...] + a1[...] + b1[...] + c1[...]) * 0.25
        ns = jnp.sum(m0 * m0, axis=1, keepdims=True) + jnp.sum(m1 * m1, axis=1, keepdims=True)
        inv = 1.0 / jnp.maximum(jnp.sqrt(ns), 1e-12)
        out[:, :_HALF] = m0 * inv
        out[:, _HALF:] = m1 * inv

    spec0 = pl.BlockSpec((blk, _HALF), lambda i: (i, 0))
    spec1 = pl.BlockSpec((blk, _HALF), lambda i: (i + half_off, 0))
    return pl.pallas_call(
        fin,
        grid=(nblk,),
        in_specs=[spec0, spec1, spec0, spec1, spec0, spec1, spec0, spec1],
        out_specs=pl.BlockSpec((blk, 2 * _HALF), lambda i: (i, 0)),
        out_shape=jax.ShapeDtypeStruct((n_pad, 2 * _HALF), jnp.float32),
    )(cat_tab, cat_tab, o1, o1, o2, o2, o3, o3)


def kernel(user_emb, item_emb, edge_vals, edge_index):
    n_users = user_emb.shape[0]
    n_nodes = n_users + item_emb.shape[0]
    n_edges = edge_vals.shape[0]
    mesh = plsc.VectorSubcoreMesh(core_axis_name="c", subcore_axis_name="s")
    n_tiles = mesh.num_subcores
    n_pad = ((n_nodes + 8 * n_tiles - 1) // (8 * n_tiles)) * (8 * n_tiles)
    e_pad = ((n_edges + n_tiles * _CHUNK - 1) // (n_tiles * _CHUNK)) * (n_tiles * _CHUNK)
    per_tile = e_pad // n_tiles

    all_emb = jnp.concatenate([user_emb, item_emb], axis=0).astype(jnp.float32)
    all_emb = jnp.pad(all_emb, ((0, n_pad - n_nodes), (0, 0)))
    cat_tab = jnp.concatenate([all_emb[:, :_HALF], all_emb[:, _HALF:]], axis=0)

    src = edge_index[0].astype(jnp.int32)
    dst = edge_index[1].astype(jnp.int32)
    vals = edge_vals.astype(jnp.float32)
    pad = e_pad - n_edges
    src = jnp.concatenate([src, jnp.zeros((pad,), jnp.int32)])
    dst = jnp.concatenate([dst, jnp.zeros((pad,), jnp.int32)])
    vals = jnp.concatenate([vals, jnp.zeros((pad,), jnp.float32)])
    src2 = jnp.stack([src, src + n_pad]).reshape(2, e_pad // 128, 128)
    dst2 = dst.reshape(e_pad // 128, 128)

    o1, o2, o3 = _sc_propagate(cat_tab, src2, dst2, vals, n_pad, per_tile)
    res = _tc_finalize(cat_tab, o1, o2, o3, n_pad)
    return res[:n_users], res[n_users:n_nodes]
